# baseline (speedup 1.0000x reference)
import functools

import jax
import jax.numpy as jnp
from jax import lax
from jax.experimental import pallas as pl
from jax.experimental.pallas import tpu as pltpu
from jax.experimental.pallas import tpu_sc as plsc

F32 = jnp.float32

_N, _E, _T, _D, _H, _G = 10000, 320000, 20, 128, 32, 128
NC, NS, L = 2, 16, 16
NW = NC * NS
NP = 10240
NPT = NP // NS
EW = 128
EP = 327680
ER = EP // EW
RPT = ER // NW

@functools.cache
def _sc_mesh():
    return plsc.VectorSubcoreMesh(
        core_axis_name="c", subcore_axis_name="s",
        num_cores=NC, num_subcores=NS)


_SC_PARAMS = pltpu.CompilerParams(needs_layout_passes=False)


def _fill(ref, n, value):
    for k in range(n // L):
        ref[pl.ds(k * L, L)] = jnp.full((L,), value, F32)


DEPTH = 8


def _deg_body(dst_hbm, out_hbm, dstv, ones_v, zbuf, deg_sh, sem):
    c = lax.axis_index("c")
    s = lax.axis_index("s")
    wid = s * NC + c
    _fill(ones_v, EW, 1.0)
    _fill(zbuf, NPT, 0.0)
    pltpu.sync_copy(zbuf, deg_sh.at[pl.ds(s * NPT, NPT)])
    pltpu.sync_copy(dst_hbm.at[pl.ds(wid * RPT, RPT)], dstv)
    plsc.subcore_barrier()
    cps = []
    for j in range(RPT):
        if j >= DEPTH:
            cps[j - DEPTH].wait()
        cps.append(
            pltpu.async_copy(ones_v, deg_sh.at[dstv.at[j]], sem, add=True))
    for j in range(RPT - DEPTH, RPT):
        cps[j].wait()
    plsc.subcore_barrier()
    pltpu.sync_copy(deg_sh.at[pl.ds(s * NPT, NPT)], zbuf)
    pltpu.sync_copy(zbuf, out_hbm.at[c, pl.ds(s * NPT, NPT)])


def _deg_call(dst2d):
    return pl.kernel(
        _deg_body,
        out_type=jax.ShapeDtypeStruct((NC, NP), F32),
        mesh=_sc_mesh(),
        compiler_params=_SC_PARAMS,
        scratch_types=[
            pltpu.VMEM((RPT, EW), jnp.int32),
            pltpu.VMEM((EW,), F32),
            pltpu.VMEM((NPT,), F32),
            pltpu.VMEM_SHARED((NP,), F32),
            pltpu.SemaphoreType.DMA,
        ],
    )(dst2d)


def _node_body(embT_ref, gwT_ref, w2T_ref, gb_ref, dp_ref, out_ref):
    w12T = jnp.dot(w2T_ref[...], gwT_ref[...], preferred_element_type=F32)
    pqT = jnp.dot(w12T, embT_ref[...], preferred_element_type=F32)
    cb = jnp.dot(w12T, gb_ref[...], preferred_element_type=F32)
    deg = dp_ref[0:1, :] + dp_ref[1:2, :] + 1.0
    dis = lax.rsqrt(deg)
    p = pqT[0:1, :]
    q = pqT[1:2, :]
    dis2 = dis * dis
    out_ref[0:1, :] = dis * p
    out_ref[1:2, :] = dis * q
    out_ref[2:3, :] = dis2 * p + cb[0:1, 0:1]
    out_ref[3:4, :] = dis2 * q + cb[1:2, 0:1]
    out_ref[4:5, :] = dis
    out_ref[5:6, :] = dis
    out_ref[6:7, :] = dis
    out_ref[7:8, :] = dis


def _node_call(embT, gwT, w2T, gb, dp):
    return pl.pallas_call(
        _node_body,
        out_shape=jax.ShapeDtypeStruct((8, NP), F32),
    )(embT, gwT, w2T, gb, dp)


def _scat_body(src_hbm, dst_hbm, pp_hbm, qq_hbm, sout, qout,
               srcv, dstv, ppv, qqv, pbufs, qbufs, zbuf, s_sh, q_sh,
               psem, qsem):
    c = lax.axis_index("c")
    s = lax.axis_index("s")
    wid = s * NC + c
    _fill(zbuf, NPT, 0.0)
    pltpu.sync_copy(zbuf, s_sh.at[pl.ds(s * NPT, NPT)])
    pltpu.sync_copy(zbuf, q_sh.at[pl.ds(s * NPT, NPT)])
    pltpu.sync_copy(pp_hbm, ppv)
    pltpu.sync_copy(qq_hbm, qqv)
    pltpu.sync_copy(src_hbm.at[pl.ds(wid * RPT, RPT)], srcv)
    pltpu.sync_copy(dst_hbm.at[pl.ds(wid * RPT, RPT)], dstv)
    plsc.subcore_barrier()
    cpp = []
    cpq = []
    for j in range(RPT):
        slot = j % DEPTH
        if j >= DEPTH:
            cpp[j - DEPTH].wait()
            cpq[j - DEPTH].wait()
        for k in range(EW // L):
            sl = pl.ds(k * L, L)
            idx = srcv[j, sl]
            pbufs[slot, sl] = plsc.load_gather(ppv, [idx])
            qbufs[slot, sl] = plsc.load_gather(qqv, [idx])
        cpp.append(pltpu.async_copy(
            pbufs.at[slot], s_sh.at[dstv.at[j]], psem, add=True))
        cpq.append(pltpu.async_copy(
            qbufs.at[slot], q_sh.at[dstv.at[j]], qsem, add=True))
    for j in range(RPT - DEPTH, RPT):
        cpp[j].wait()
        cpq[j].wait()
    plsc.subcore_barrier()
    pltpu.sync_copy(s_sh.at[pl.ds(s * NPT, NPT)], zbuf)
    pltpu.sync_copy(zbuf, sout.at[c, pl.ds(s * NPT, NPT)])
    pltpu.sync_copy(q_sh.at[pl.ds(s * NPT, NPT)], zbuf)
    pltpu.sync_copy(zbuf, qout.at[c, pl.ds(s * NPT, NPT)])


def _scat_call(src2d, dst2d, pp, qq):
    return pl.kernel(
        _scat_body,
        out_type=(
            jax.ShapeDtypeStruct((NC, NP), F32),
            jax.ShapeDtypeStruct((NC, NP), F32),
        ),
        mesh=_sc_mesh(),
        compiler_params=_SC_PARAMS,
        scratch_types=[
            pltpu.VMEM((RPT, EW), jnp.int32),
            pltpu.VMEM((RPT, EW), jnp.int32),
            pltpu.VMEM((NP,), F32),
            pltpu.VMEM((NP,), F32),
            pltpu.VMEM((DEPTH, EW), F32),
            pltpu.VMEM((DEPTH, EW), F32),
            pltpu.VMEM((NPT,), F32),
            pltpu.VMEM_SHARED((NP,), F32),
            pltpu.VMEM_SHARED((NP,), F32),
            pltpu.SemaphoreType.DMA,
            pltpu.SemaphoreType.DMA,
        ],
    )(src2d, dst2d, pp, qq)


EB = 5120
KA = 40
NCHUNK = 4
EBC = EB // NCHUNK


BF16 = jnp.bfloat16


def _lstm_body(x_ref, waug_ref, wl_ref, pb_ref, out_ref, haug_ref):
    xb = x_ref[...]
    waug = waug_ref[...]
    sub = lax.broadcasted_iota(jnp.int32, (8, EB), 0)
    haug_ref[_H:_H + 8, :] = jnp.where(sub == 1, 1.0, 0.0).astype(BF16)
    haug_ref[0:_H, :] = jnp.zeros((_H, EB), BF16)
    ccs = [jnp.zeros((_H, EBC), F32) for _ in range(NCHUNK)]
    for t in range(_T):
        haug_ref[_H:_H + 1, :] = xb[t:t + 1, :]
        for ch in range(NCHUNK):
            sl = pl.ds(ch * EBC, EBC)
            g = jnp.dot(waug, haug_ref[:, sl], preferred_element_type=F32)
            ti = jnp.tanh(g[0 * _H:1 * _H, :])
            tf = jnp.tanh(g[1 * _H:2 * _H, :])
            tg = jnp.tanh(g[2 * _H:3 * _H, :])
            to = jnp.tanh(g[3 * _H:4 * _H, :])
            cc = 0.5 * ((tf + 1.0) * ccs[ch] + (ti + 1.0) * tg)
            ccs[ch] = cc
            haug_ref[0:_H, sl] = ((to + 1.0) * jnp.tanh(cc)).astype(BF16)
    res = jnp.dot(wl_ref[...], haug_ref[0:_H, :],
                  preferred_element_type=F32)
    out_ref[...] = res[0:1, :] + pb_ref[...]


def _lstm_call(xT, waug, wl8, pb):
    grid = (EP // EB,)
    return pl.pallas_call(
        _lstm_body,
        grid=grid,
        in_specs=[
            pl.BlockSpec((_T, EB), lambda j: (0, j)),
            pl.BlockSpec((4 * _H, KA), lambda j: (0, 0)),
            pl.BlockSpec((8, _H), lambda j: (0, 0)),
            pl.BlockSpec((1, 1), lambda j: (0, 0)),
        ],
        out_specs=pl.BlockSpec((1, EB), lambda j: (0, j)),
        out_shape=jax.ShapeDtypeStruct((1, EP), F32),
        scratch_shapes=[pltpu.VMEM((KA, EB), BF16)],
    )(xT, waug, wl8, pb)


def _final_body(src_hbm, dst_hbm, sacc_hbm, qacc_hbm, dis_hbm, bs_hbm, bd_hbm,
                l_hbm, out_hbm,
                srcv, dstv, lv, obuf, sn_v, dn_v, t0, t1, t2, t3, sn_sh, dn_sh):
    c = lax.axis_index("c")
    s = lax.axis_index("s")
    wid = s * NC + c
    base = s * NPT
    pltpu.sync_copy(sacc_hbm.at[0, pl.ds(base, NPT)], t0)
    pltpu.sync_copy(sacc_hbm.at[1, pl.ds(base, NPT)], t1)
    pltpu.sync_copy(dis_hbm.at[pl.ds(base, NPT)], t2)
    pltpu.sync_copy(bs_hbm.at[pl.ds(base, NPT)], t3)
    for k in range(NPT // L):
        sl = pl.ds(k * L, L)
        t0[sl] = t2[sl] * (t0[sl] + t1[sl]) + t3[sl]
    pltpu.sync_copy(t0, sn_sh.at[pl.ds(base, NPT)])
    pltpu.sync_copy(qacc_hbm.at[0, pl.ds(base, NPT)], t0)
    pltpu.sync_copy(qacc_hbm.at[1, pl.ds(base, NPT)], t1)
    pltpu.sync_copy(bd_hbm.at[pl.ds(base, NPT)], t3)
    for k in range(NPT // L):
        sl = pl.ds(k * L, L)
        t0[sl] = t2[sl] * (t0[sl] + t1[sl]) + t3[sl]
    pltpu.sync_copy(t0, dn_sh.at[pl.ds(base, NPT)])
    plsc.subcore_barrier()
    pltpu.sync_copy(sn_sh, sn_v)
    pltpu.sync_copy(dn_sh, dn_v)
    pltpu.sync_copy(src_hbm.at[pl.ds(wid * RPT, RPT)], srcv)
    pltpu.sync_copy(dst_hbm.at[pl.ds(wid * RPT, RPT)], dstv)
    pltpu.sync_copy(l_hbm.at[pl.ds(wid * RPT, RPT)], lv)

    for j in range(RPT):
        for k in range(EW // L):
            sl = pl.ds(k * L, L)
            si = srcv[j, sl]
            di = dstv[j, sl]
            obuf[j, sl] = (plsc.load_gather(sn_v, [si])
                           + plsc.load_gather(dn_v, [di]) + lv[j, sl])
    pltpu.sync_copy(obuf, out_hbm.at[pl.ds(wid * RPT, RPT)])


def _final_call(src2d, dst2d, sacc, qacc, dis, bs, bd, l2d):
    return pl.kernel(
        _final_body,
        out_type=jax.ShapeDtypeStruct((ER, EW), F32),
        mesh=_sc_mesh(),
        compiler_params=_SC_PARAMS,
        scratch_types=[
            pltpu.VMEM((RPT, EW), jnp.int32),
            pltpu.VMEM((RPT, EW), jnp.int32),
            pltpu.VMEM((RPT, EW), F32),
            pltpu.VMEM((RPT, EW), F32),
            pltpu.VMEM((NP,), F32),
            pltpu.VMEM((NP,), F32),
            pltpu.VMEM((NPT,), F32),
            pltpu.VMEM((NPT,), F32),
            pltpu.VMEM((NPT,), F32),
            pltpu.VMEM((NPT,), F32),
            pltpu.VMEM_SHARED((NP,), F32),
            pltpu.VMEM_SHARED((NP,), F32),
        ],
    )(src2d, dst2d, sacc, qacc, dis, bs, bd, l2d)


def kernel(x, edge_index, emb, gcn_W, gcn_b, W_ih, W_hh, b_ih, b_hh,
           pred_W, pred_b):
    epad = jnp.full((EP - _E,), _N, edge_index.dtype)
    src2d = jnp.concatenate([edge_index[0], epad]).reshape(ER, EW)
    dst2d = jnp.concatenate([edge_index[1], epad]).reshape(ER, EW)

    deg_part = _deg_call(dst2d)

    embT = jnp.pad(emb, ((0, NP - _N), (0, 0))).T
    w2T = (jnp.zeros((8, _D), F32)
           .at[0].set(pred_W[0, :_G])
           .at[1].set(pred_W[0, _G:2 * _G]))
    nodes = _node_call(embT, gcn_W.T, w2T, gcn_b.reshape(_D, 1), deg_part)
    pp, qq, bs, bd, dis = (nodes[0], nodes[1], nodes[2], nodes[3], nodes[4])

    sacc, qacc = _scat_call(src2d, dst2d, pp, qq)

    bias = (b_ih + b_hh).reshape(4 * _H, 1)
    row_scale = jnp.concatenate([
        jnp.full((2 * _H, 1), 0.5, F32),
        jnp.ones((_H, 1), F32),
        jnp.full((_H, 1), 0.5, F32),
    ])
    waug = jnp.concatenate([W_hh * 0.5, W_ih, bias], axis=1) * row_scale
    waug = jnp.pad(waug, ((0, 0), (0, KA - (_H + 2)))).astype(BF16)
    wl8 = (jnp.zeros((8, _H), F32)
           .at[0].set(pred_W[0, 2 * _G:] * 0.5)).astype(BF16)
    xT = jnp.pad(x.T, ((0, 0), (0, EP - _E))).astype(BF16)
    lstm = _lstm_call(xT, waug, wl8, pred_b.reshape(1, 1))
    l2d = lstm.reshape(ER, EW)

    out2d = _final_call(src2d, dst2d, sacc, qacc, dis, bs, bd, l2d)
    return out2d.reshape(EP)[:_E]


# speedup vs baseline: 1.0116x; 1.0116x over previous
import functools

import jax
import jax.numpy as jnp
from jax import lax
from jax.experimental import pallas as pl
from jax.experimental.pallas import tpu as pltpu
from jax.experimental.pallas import tpu_sc as plsc

F32 = jnp.float32

_N, _E, _T, _D, _H, _G = 10000, 320000, 20, 128, 32, 128
NC, NS, L = 2, 16, 16
NW = NC * NS
NP = 10240
NPT = NP // NS
EW = 128
EP = 327680
ER = EP // EW
RPT = ER // NW

@functools.cache
def _sc_mesh():
    return plsc.VectorSubcoreMesh(
        core_axis_name="c", subcore_axis_name="s",
        num_cores=NC, num_subcores=NS)


_SC_PARAMS = pltpu.CompilerParams(needs_layout_passes=False)


def _fill(ref, n, value):
    for k in range(n // L):
        ref[pl.ds(k * L, L)] = jnp.full((L,), value, F32)


DEPTH = 8


def _deg_body(dst_hbm, out_hbm, dstv, ones_v, zbuf, deg_sh, sem):
    c = lax.axis_index("c")
    s = lax.axis_index("s")
    wid = s * NC + c
    _fill(ones_v, EW, 1.0)
    _fill(zbuf, NPT, 0.0)
    pltpu.sync_copy(zbuf, deg_sh.at[pl.ds(s * NPT, NPT)])
    pltpu.sync_copy(dst_hbm.at[pl.ds(wid * RPT, RPT)], dstv)
    plsc.subcore_barrier()
    cps = []
    for j in range(RPT):
        if j >= DEPTH:
            cps[j - DEPTH].wait()
        cps.append(
            pltpu.async_copy(ones_v, deg_sh.at[dstv.at[j]], sem, add=True))
    for j in range(RPT - DEPTH, RPT):
        cps[j].wait()
    plsc.subcore_barrier()
    pltpu.sync_copy(deg_sh.at[pl.ds(s * NPT, NPT)], zbuf)
    pltpu.sync_copy(zbuf, out_hbm.at[c, pl.ds(s * NPT, NPT)])


def _deg_call(dst2d):
    return pl.kernel(
        _deg_body,
        out_type=jax.ShapeDtypeStruct((NC, NP), F32),
        mesh=_sc_mesh(),
        compiler_params=_SC_PARAMS,
        scratch_types=[
            pltpu.VMEM((RPT, EW), jnp.int32),
            pltpu.VMEM((EW,), F32),
            pltpu.VMEM((NPT,), F32),
            pltpu.VMEM_SHARED((NP,), F32),
            pltpu.SemaphoreType.DMA,
        ],
    )(dst2d)


def _node_body(embT_ref, gwT_ref, w2T_ref, gb_ref, dp_ref, out_ref):
    w12T = jnp.dot(w2T_ref[...], gwT_ref[...], preferred_element_type=F32)
    pqT = jnp.dot(w12T, embT_ref[...], preferred_element_type=F32)
    cb = jnp.dot(w12T, gb_ref[...], preferred_element_type=F32)
    deg = dp_ref[0:1, :] + dp_ref[1:2, :] + 1.0
    dis = lax.rsqrt(deg)
    p = pqT[0:1, :]
    q = pqT[1:2, :]
    dis2 = dis * dis
    out_ref[0:1, :] = dis * p
    out_ref[1:2, :] = dis * q
    out_ref[2:3, :] = dis2 * p + cb[0:1, 0:1]
    out_ref[3:4, :] = dis2 * q + cb[1:2, 0:1]
    out_ref[4:5, :] = dis
    out_ref[5:6, :] = dis
    out_ref[6:7, :] = dis
    out_ref[7:8, :] = dis


def _node_call(embT, gwT, w2T, gb, dp):
    return pl.pallas_call(
        _node_body,
        out_shape=jax.ShapeDtypeStruct((8, NP), F32),
    )(embT, gwT, w2T, gb, dp)


def _scat_body(src_hbm, dst_hbm, pp_hbm, qq_hbm, sout, qout,
               srcv, dstv, ppv, qqv, pbufs, qbufs, zbuf, s_sh, q_sh,
               psem, qsem):
    c = lax.axis_index("c")
    s = lax.axis_index("s")
    wid = s * NC + c
    _fill(zbuf, NPT, 0.0)
    pltpu.sync_copy(zbuf, s_sh.at[pl.ds(s * NPT, NPT)])
    pltpu.sync_copy(zbuf, q_sh.at[pl.ds(s * NPT, NPT)])
    pltpu.sync_copy(pp_hbm, ppv)
    pltpu.sync_copy(qq_hbm, qqv)
    pltpu.sync_copy(src_hbm.at[pl.ds(wid * RPT, RPT)], srcv)
    pltpu.sync_copy(dst_hbm.at[pl.ds(wid * RPT, RPT)], dstv)
    plsc.subcore_barrier()
    cpp = []
    cpq = []
    for j in range(RPT):
        slot = j % DEPTH
        if j >= DEPTH:
            cpp[j - DEPTH].wait()
            cpq[j - DEPTH].wait()
        for k in range(EW // L):
            sl = pl.ds(k * L, L)
            idx = srcv[j, sl]
            pbufs[slot, sl] = plsc.load_gather(ppv, [idx])
            qbufs[slot, sl] = plsc.load_gather(qqv, [idx])
        cpp.append(pltpu.async_copy(
            pbufs.at[slot], s_sh.at[dstv.at[j]], psem, add=True))
        cpq.append(pltpu.async_copy(
            qbufs.at[slot], q_sh.at[dstv.at[j]], qsem, add=True))
    for j in range(RPT - DEPTH, RPT):
        cpp[j].wait()
        cpq[j].wait()
    plsc.subcore_barrier()
    pltpu.sync_copy(s_sh.at[pl.ds(s * NPT, NPT)], zbuf)
    pltpu.sync_copy(zbuf, sout.at[c, pl.ds(s * NPT, NPT)])
    pltpu.sync_copy(q_sh.at[pl.ds(s * NPT, NPT)], zbuf)
    pltpu.sync_copy(zbuf, qout.at[c, pl.ds(s * NPT, NPT)])


def _scat_call(src2d, dst2d, pp, qq):
    return pl.kernel(
        _scat_body,
        out_type=(
            jax.ShapeDtypeStruct((NC, NP), F32),
            jax.ShapeDtypeStruct((NC, NP), F32),
        ),
        mesh=_sc_mesh(),
        compiler_params=_SC_PARAMS,
        scratch_types=[
            pltpu.VMEM((RPT, EW), jnp.int32),
            pltpu.VMEM((RPT, EW), jnp.int32),
            pltpu.VMEM((NP,), F32),
            pltpu.VMEM((NP,), F32),
            pltpu.VMEM((DEPTH, EW), F32),
            pltpu.VMEM((DEPTH, EW), F32),
            pltpu.VMEM((NPT,), F32),
            pltpu.VMEM_SHARED((NP,), F32),
            pltpu.VMEM_SHARED((NP,), F32),
            pltpu.SemaphoreType.DMA,
            pltpu.SemaphoreType.DMA,
        ],
    )(src2d, dst2d, pp, qq)


EB = 5120
KA = 40
NCHUNK = 4
EBC = EB // NCHUNK


BF16 = jnp.bfloat16


def _lstm_body(x_ref, waug_ref, wl_ref, pb_ref, out_ref, haug_ref):
    xb = x_ref[...]
    waug = waug_ref[...]
    sub = lax.broadcasted_iota(jnp.int32, (8, EB), 0)
    haug_ref[_H:_H + 8, :] = jnp.where(sub == 1, 1.0, 0.0).astype(BF16)
    haug_ref[0:_H, :] = jnp.zeros((_H, EB), BF16)
    ccs = [jnp.zeros((_H, EBC), F32) for _ in range(NCHUNK)]
    for t in range(_T):
        haug_ref[_H:_H + 1, :] = xb[t:t + 1, :]
        for ch in range(NCHUNK):
            sl = pl.ds(ch * EBC, EBC)
            g = jnp.dot(waug, haug_ref[:, sl], preferred_element_type=F32)
            ti = jnp.tanh(g[0 * _H:1 * _H, :])
            tf = jnp.tanh(g[1 * _H:2 * _H, :])
            tg = jnp.tanh(g[2 * _H:3 * _H, :])
            to = jnp.tanh(g[3 * _H:4 * _H, :])
            cc = 0.5 * ((tf + 1.0) * ccs[ch] + (ti + 1.0) * tg)
            ccs[ch] = cc
            haug_ref[0:_H, sl] = ((to + 1.0) * jnp.tanh(cc)).astype(BF16)
    res = jnp.dot(wl_ref[...], haug_ref[0:_H, :],
                  preferred_element_type=F32)
    out_ref[...] = res[0:1, :] + pb_ref[...]


def _lstm_call(xT, waug, wl8, pb):
    grid = (EP // EB,)
    return pl.pallas_call(
        _lstm_body,
        grid=grid,
        in_specs=[
            pl.BlockSpec((_T, EB), lambda j: (0, j)),
            pl.BlockSpec((4 * _H, KA), lambda j: (0, 0)),
            pl.BlockSpec((8, _H), lambda j: (0, 0)),
            pl.BlockSpec((1, 1), lambda j: (0, 0)),
        ],
        out_specs=pl.BlockSpec((1, EB), lambda j: (0, j)),
        out_shape=jax.ShapeDtypeStruct((1, EP), F32),
        scratch_shapes=[pltpu.VMEM((KA, EB), BF16)],
    )(xT, waug, wl8, pb)


def _final_body(src_hbm, dst_hbm, sacc_hbm, qacc_hbm, dis_hbm, bs_hbm, bd_hbm,
                l_hbm, out_hbm,
                srcv, dstv, lv, obuf, sn_v, dn_v, t0, t1, t2, t3, sn_sh, dn_sh):
    c = lax.axis_index("c")
    s = lax.axis_index("s")
    wid = s * NC + c
    base = s * NPT
    pltpu.sync_copy(sacc_hbm.at[0, pl.ds(base, NPT)], t0)
    pltpu.sync_copy(sacc_hbm.at[1, pl.ds(base, NPT)], t1)
    pltpu.sync_copy(dis_hbm.at[pl.ds(base, NPT)], t2)
    pltpu.sync_copy(bs_hbm.at[pl.ds(base, NPT)], t3)
    for k in range(NPT // L):
        sl = pl.ds(k * L, L)
        t0[sl] = t2[sl] * (t0[sl] + t1[sl]) + t3[sl]
    pltpu.sync_copy(t0, sn_sh.at[pl.ds(base, NPT)])
    pltpu.sync_copy(qacc_hbm.at[0, pl.ds(base, NPT)], t0)
    pltpu.sync_copy(qacc_hbm.at[1, pl.ds(base, NPT)], t1)
    pltpu.sync_copy(bd_hbm.at[pl.ds(base, NPT)], t3)
    for k in range(NPT // L):
        sl = pl.ds(k * L, L)
        t0[sl] = t2[sl] * (t0[sl] + t1[sl]) + t3[sl]
    pltpu.sync_copy(t0, dn_sh.at[pl.ds(base, NPT)])
    plsc.subcore_barrier()
    pltpu.sync_copy(sn_sh, sn_v)
    pltpu.sync_copy(dn_sh, dn_v)
    pltpu.sync_copy(src_hbm.at[pl.ds(wid * RPT, RPT)], srcv)
    pltpu.sync_copy(dst_hbm.at[pl.ds(wid * RPT, RPT)], dstv)
    pltpu.sync_copy(l_hbm.at[pl.ds(wid * RPT, RPT)], lv)

    def body(j, carry):
        for k in range(EW // L):
            sl = pl.ds(k * L, L)
            si = srcv[j, sl]
            di = dstv[j, sl]
            obuf[j, sl] = (plsc.load_gather(sn_v, [si])
                           + plsc.load_gather(dn_v, [di]) + lv[j, sl])
        return carry

    lax.fori_loop(0, RPT, body, 0)
    pltpu.sync_copy(obuf, out_hbm.at[pl.ds(wid * RPT, RPT)])


def _final_call(src2d, dst2d, sacc, qacc, dis, bs, bd, l2d):
    return pl.kernel(
        _final_body,
        out_type=jax.ShapeDtypeStruct((ER, EW), F32),
        mesh=_sc_mesh(),
        compiler_params=_SC_PARAMS,
        scratch_types=[
            pltpu.VMEM((RPT, EW), jnp.int32),
            pltpu.VMEM((RPT, EW), jnp.int32),
            pltpu.VMEM((RPT, EW), F32),
            pltpu.VMEM((RPT, EW), F32),
            pltpu.VMEM((NP,), F32),
            pltpu.VMEM((NP,), F32),
            pltpu.VMEM((NPT,), F32),
            pltpu.VMEM((NPT,), F32),
            pltpu.VMEM((NPT,), F32),
            pltpu.VMEM((NPT,), F32),
            pltpu.VMEM_SHARED((NP,), F32),
            pltpu.VMEM_SHARED((NP,), F32),
        ],
    )(src2d, dst2d, sacc, qacc, dis, bs, bd, l2d)


def kernel(x, edge_index, emb, gcn_W, gcn_b, W_ih, W_hh, b_ih, b_hh,
           pred_W, pred_b):
    epad = jnp.full((EP - _E,), _N, edge_index.dtype)
    src2d = jnp.concatenate([edge_index[0], epad]).reshape(ER, EW)
    dst2d = jnp.concatenate([edge_index[1], epad]).reshape(ER, EW)

    deg_part = _deg_call(dst2d)

    embT = jnp.pad(emb, ((0, NP - _N), (0, 0))).T
    w2T = (jnp.zeros((8, _D), F32)
           .at[0].set(pred_W[0, :_G])
           .at[1].set(pred_W[0, _G:2 * _G]))
    nodes = _node_call(embT, gcn_W.T, w2T, gcn_b.reshape(_D, 1), deg_part)
    pp, qq, bs, bd, dis = (nodes[0], nodes[1], nodes[2], nodes[3], nodes[4])

    sacc, qacc = _scat_call(src2d, dst2d, pp, qq)

    bias = (b_ih + b_hh).reshape(4 * _H, 1)
    row_scale = jnp.concatenate([
        jnp.full((2 * _H, 1), 0.5, F32),
        jnp.ones((_H, 1), F32),
        jnp.full((_H, 1), 0.5, F32),
    ])
    waug = jnp.concatenate([W_hh * 0.5, W_ih, bias], axis=1) * row_scale
    waug = jnp.pad(waug, ((0, 0), (0, KA - (_H + 2)))).astype(BF16)
    wl8 = (jnp.zeros((8, _H), F32)
           .at[0].set(pred_W[0, 2 * _G:] * 0.5)).astype(BF16)
    xT = jnp.pad(x.T, ((0, 0), (0, EP - _E))).astype(BF16)
    lstm = _lstm_call(xT, waug, wl8, pred_b.reshape(1, 1))
    l2d = lstm.reshape(ER, EW)

    out2d = _final_call(src2d, dst2d, sacc, qacc, dis, bs, bd, l2d)
    return out2d.reshape(EP)[:_E]


# speedup vs baseline: 1.0223x; 1.0106x over previous
import functools

import jax
import jax.numpy as jnp
from jax import lax
from jax.experimental import pallas as pl
from jax.experimental.pallas import tpu as pltpu
from jax.experimental.pallas import tpu_sc as plsc

F32 = jnp.float32

_N, _E, _T, _D, _H, _G = 10000, 320000, 20, 128, 32, 128
NC, NS, L = 2, 16, 16
NW = NC * NS
NP = 10240
NPT = NP // NS
EW = 128
EP = 327680
ER = EP // EW
RPT = ER // NW

@functools.cache
def _sc_mesh():
    return plsc.VectorSubcoreMesh(
        core_axis_name="c", subcore_axis_name="s",
        num_cores=NC, num_subcores=NS)


_SC_PARAMS = pltpu.CompilerParams(needs_layout_passes=False)


def _fill(ref, n, value):
    for k in range(n // L):
        ref[pl.ds(k * L, L)] = jnp.full((L,), value, F32)


DEPTH = 8
ERS = ER // NS


def _node_body(embT_ref, gwT_ref, w2T_ref, gb_ref, out_ref):
    w12T = jnp.dot(w2T_ref[...], gwT_ref[...], preferred_element_type=F32)
    pqT = jnp.dot(w12T, embT_ref[...], preferred_element_type=F32)
    cb = jnp.dot(w12T, gb_ref[...], preferred_element_type=F32)
    zero = jnp.zeros((1, NP), F32)
    out_ref[0:1, :] = pqT[0:1, :]
    out_ref[1:2, :] = pqT[1:2, :]
    out_ref[2:3, :] = cb[0:1, 0:1] + zero
    out_ref[3:4, :] = cb[1:2, 0:1] + zero
    out_ref[4:5, :] = zero
    out_ref[5:6, :] = zero
    out_ref[6:7, :] = zero
    out_ref[7:8, :] = zero


def _node_call(embT, gwT, w2T, gb):
    return pl.pallas_call(
        _node_body,
        out_shape=jax.ShapeDtypeStruct((8, NP), F32),
    )(embT, gwT, w2T, gb)


def _rsqrt16(x):
    i = plsc.bitcast(x, jnp.int32)
    i = jnp.int32(0x5F3759DF) - lax.shift_right_logical(i, 1)
    y = plsc.bitcast(i, F32)
    for _ in range(3):
        y = y * (1.5 - 0.5 * x * y * y)
    return y


def _scat_body(src_hbm, dst_hbm, p_hbm, q_hbm, sout, qout, disout,
               srcv, dstv, dstcnt, ppv, qqv, pbufs, qbufs, ones_v, zbuf,
               tdis, tmp, deg_sh, s_sh, q_sh, pp_sh, qq_sh, psem, qsem):
    c = lax.axis_index("c")
    s = lax.axis_index("s")
    wid = s * NC + c
    sl_n = pl.ds(s * NPT, NPT)
    _fill(ones_v, EW, 1.0)
    _fill(zbuf, NPT, 0.0)
    pltpu.sync_copy(zbuf, deg_sh.at[sl_n])
    pltpu.sync_copy(zbuf, s_sh.at[sl_n])
    pltpu.sync_copy(zbuf, q_sh.at[sl_n])
    pltpu.sync_copy(dst_hbm.at[pl.ds(s * ERS, ERS)], dstcnt)
    plsc.subcore_barrier()
    cps = []
    for j in range(ERS):
        if j >= DEPTH:
            cps[j - DEPTH].wait()
        cps.append(
            pltpu.async_copy(ones_v, deg_sh.at[dstcnt.at[j]], psem, add=True))
    for j in range(ERS - DEPTH, ERS):
        cps[j].wait()
    plsc.subcore_barrier()
    pltpu.sync_copy(deg_sh.at[sl_n], tdis)
    pltpu.sync_copy(p_hbm.at[sl_n], tmp)
    for k in range(NPT // L):
        sl = pl.ds(k * L, L)
        y = _rsqrt16(tdis[sl] + 1.0)
        tdis[sl] = y
        tmp[sl] = y * tmp[sl]
    pltpu.sync_copy(tmp, pp_sh.at[sl_n])

    @pl.when(c == 0)
    def _():
        pltpu.sync_copy(tdis, disout.at[sl_n])

    pltpu.sync_copy(q_hbm.at[sl_n], tmp)
    for k in range(NPT // L):
        sl = pl.ds(k * L, L)
        tmp[sl] = tdis[sl] * tmp[sl]
    pltpu.sync_copy(tmp, qq_sh.at[sl_n])
    plsc.subcore_barrier()
    pltpu.sync_copy(pp_sh, ppv)
    pltpu.sync_copy(qq_sh, qqv)
    pltpu.sync_copy(src_hbm.at[pl.ds(wid * RPT, RPT)], srcv)
    pltpu.sync_copy(dst_hbm.at[pl.ds(wid * RPT, RPT)], dstv)
    cpp = []
    cpq = []
    for j in range(RPT):
        slot = j % DEPTH
        if j >= DEPTH:
            cpp[j - DEPTH].wait()
            cpq[j - DEPTH].wait()
        for k in range(EW // L):
            sl = pl.ds(k * L, L)
            idx = srcv[j, sl]
            pbufs[slot, sl] = plsc.load_gather(ppv, [idx])
            qbufs[slot, sl] = plsc.load_gather(qqv, [idx])
        cpp.append(pltpu.async_copy(
            pbufs.at[slot], s_sh.at[dstv.at[j]], psem, add=True))
        cpq.append(pltpu.async_copy(
            qbufs.at[slot], q_sh.at[dstv.at[j]], qsem, add=True))
    for j in range(RPT - DEPTH, RPT):
        cpp[j].wait()
        cpq[j].wait()
    plsc.subcore_barrier()
    pltpu.sync_copy(s_sh.at[sl_n], zbuf)
    pltpu.sync_copy(zbuf, sout.at[c, sl_n])
    pltpu.sync_copy(q_sh.at[sl_n], zbuf)
    pltpu.sync_copy(zbuf, qout.at[c, sl_n])


def _scat_call(src2d, dst2d, p, q):
    return pl.kernel(
        _scat_body,
        out_type=(
            jax.ShapeDtypeStruct((NC, NP), F32),
            jax.ShapeDtypeStruct((NC, NP), F32),
            jax.ShapeDtypeStruct((NP,), F32),
        ),
        mesh=_sc_mesh(),
        compiler_params=_SC_PARAMS,
        scratch_types=[
            pltpu.VMEM((RPT, EW), jnp.int32),
            pltpu.VMEM((RPT, EW), jnp.int32),
            pltpu.VMEM((ERS, EW), jnp.int32),
            pltpu.VMEM((NP,), F32),
            pltpu.VMEM((NP,), F32),
            pltpu.VMEM((DEPTH, EW), F32),
            pltpu.VMEM((DEPTH, EW), F32),
            pltpu.VMEM((EW,), F32),
            pltpu.VMEM((NPT,), F32),
            pltpu.VMEM((NPT,), F32),
            pltpu.VMEM((NPT,), F32),
            pltpu.VMEM_SHARED((NP,), F32),
            pltpu.VMEM_SHARED((NP,), F32),
            pltpu.VMEM_SHARED((NP,), F32),
            pltpu.VMEM_SHARED((NP,), F32),
            pltpu.VMEM_SHARED((NP,), F32),
            pltpu.SemaphoreType.DMA,
            pltpu.SemaphoreType.DMA,
        ],
    )(src2d, dst2d, p, q)


EB = 5120
KA = 40
NCHUNK = 4
EBC = EB // NCHUNK


BF16 = jnp.bfloat16


def _lstm_body(x_ref, waug_ref, wl_ref, pb_ref, out_ref, haug_ref):
    xb = x_ref[...]
    waug = waug_ref[...]
    sub = lax.broadcasted_iota(jnp.int32, (8, EB), 0)
    haug_ref[_H:_H + 8, :] = jnp.where(sub == 1, 1.0, 0.0).astype(BF16)
    haug_ref[0:_H, :] = jnp.zeros((_H, EB), BF16)
    ccs = [jnp.zeros((_H, EBC), F32) for _ in range(NCHUNK)]
    for t in range(_T):
        haug_ref[_H:_H + 1, :] = xb[t:t + 1, :]
        for ch in range(NCHUNK):
            sl = pl.ds(ch * EBC, EBC)
            g = jnp.dot(waug, haug_ref[:, sl], preferred_element_type=F32)
            ti = jnp.tanh(g[0 * _H:1 * _H, :])
            tf = jnp.tanh(g[1 * _H:2 * _H, :])
            tg = jnp.tanh(g[2 * _H:3 * _H, :])
            to = jnp.tanh(g[3 * _H:4 * _H, :])
            cc = 0.5 * ((tf + 1.0) * ccs[ch] + (ti + 1.0) * tg)
            ccs[ch] = cc
            haug_ref[0:_H, sl] = ((to + 1.0) * jnp.tanh(cc)).astype(BF16)
    res = jnp.dot(wl_ref[...], haug_ref[0:_H, :],
                  preferred_element_type=F32)
    out_ref[...] = res[0:1, :] + pb_ref[...]


def _lstm_call(xT, waug, wl8, pb):
    grid = (EP // EB,)
    return pl.pallas_call(
        _lstm_body,
        grid=grid,
        in_specs=[
            pl.BlockSpec((_T, EB), lambda j: (0, j)),
            pl.BlockSpec((4 * _H, KA), lambda j: (0, 0)),
            pl.BlockSpec((8, _H), lambda j: (0, 0)),
            pl.BlockSpec((1, 1), lambda j: (0, 0)),
        ],
        out_specs=pl.BlockSpec((1, EB), lambda j: (0, j)),
        out_shape=jax.ShapeDtypeStruct((1, EP), F32),
        scratch_shapes=[pltpu.VMEM((KA, EB), BF16)],
    )(xT, waug, wl8, pb)


def _final_body(src_hbm, dst_hbm, sacc_hbm, qacc_hbm, dis_hbm, p_hbm, q_hbm,
                cbs_hbm, cbd_hbm, l_hbm, out_hbm,
                srcv, dstv, lv, obuf, sn_v, dn_v, t0, t1, t2, t3, t4,
                sn_sh, dn_sh):
    c = lax.axis_index("c")
    s = lax.axis_index("s")
    wid = s * NC + c
    base = s * NPT
    pltpu.sync_copy(sacc_hbm.at[0, pl.ds(base, NPT)], t0)
    pltpu.sync_copy(sacc_hbm.at[1, pl.ds(base, NPT)], t1)
    pltpu.sync_copy(dis_hbm.at[pl.ds(base, NPT)], t2)
    pltpu.sync_copy(p_hbm.at[pl.ds(base, NPT)], t3)
    pltpu.sync_copy(cbs_hbm.at[pl.ds(base, NPT)], t4)
    for k in range(NPT // L):
        sl = pl.ds(k * L, L)
        t0[sl] = t2[sl] * (t0[sl] + t1[sl] + t2[sl] * t3[sl]) + t4[sl]
    pltpu.sync_copy(t0, sn_sh.at[pl.ds(base, NPT)])
    pltpu.sync_copy(qacc_hbm.at[0, pl.ds(base, NPT)], t0)
    pltpu.sync_copy(qacc_hbm.at[1, pl.ds(base, NPT)], t1)
    pltpu.sync_copy(q_hbm.at[pl.ds(base, NPT)], t3)
    pltpu.sync_copy(cbd_hbm.at[pl.ds(base, NPT)], t4)
    for k in range(NPT // L):
        sl = pl.ds(k * L, L)
        t0[sl] = t2[sl] * (t0[sl] + t1[sl] + t2[sl] * t3[sl]) + t4[sl]
    pltpu.sync_copy(t0, dn_sh.at[pl.ds(base, NPT)])
    plsc.subcore_barrier()
    pltpu.sync_copy(sn_sh, sn_v)
    pltpu.sync_copy(dn_sh, dn_v)
    pltpu.sync_copy(src_hbm.at[pl.ds(wid * RPT, RPT)], srcv)
    pltpu.sync_copy(dst_hbm.at[pl.ds(wid * RPT, RPT)], dstv)
    pltpu.sync_copy(l_hbm.at[pl.ds(wid * RPT, RPT)], lv)

    def body(j, carry):
        for k in range(EW // L):
            sl = pl.ds(k * L, L)
            si = srcv[j, sl]
            di = dstv[j, sl]
            obuf[j, sl] = (plsc.load_gather(sn_v, [si])
                           + plsc.load_gather(dn_v, [di]) + lv[j, sl])
        return carry

    lax.fori_loop(0, RPT, body, 0)
    pltpu.sync_copy(obuf, out_hbm.at[pl.ds(wid * RPT, RPT)])


def _final_call(src2d, dst2d, sacc, qacc, dis, p, q, cbs, cbd, l2d):
    return pl.kernel(
        _final_body,
        out_type=jax.ShapeDtypeStruct((ER, EW), F32),
        mesh=_sc_mesh(),
        compiler_params=_SC_PARAMS,
        scratch_types=[
            pltpu.VMEM((RPT, EW), jnp.int32),
            pltpu.VMEM((RPT, EW), jnp.int32),
            pltpu.VMEM((RPT, EW), F32),
            pltpu.VMEM((RPT, EW), F32),
            pltpu.VMEM((NP,), F32),
            pltpu.VMEM((NP,), F32),
            pltpu.VMEM((NPT,), F32),
            pltpu.VMEM((NPT,), F32),
            pltpu.VMEM((NPT,), F32),
            pltpu.VMEM((NPT,), F32),
            pltpu.VMEM((NPT,), F32),
            pltpu.VMEM_SHARED((NP,), F32),
            pltpu.VMEM_SHARED((NP,), F32),
        ],
    )(src2d, dst2d, sacc, qacc, dis, p, q, cbs, cbd, l2d)


def kernel(x, edge_index, emb, gcn_W, gcn_b, W_ih, W_hh, b_ih, b_hh,
           pred_W, pred_b):
    epad = jnp.full((EP - _E,), _N, edge_index.dtype)
    src2d = jnp.concatenate([edge_index[0], epad]).reshape(ER, EW)
    dst2d = jnp.concatenate([edge_index[1], epad]).reshape(ER, EW)

    embT = jnp.pad(emb, ((0, NP - _N), (0, 0))).T
    w2T = (jnp.zeros((8, _D), F32)
           .at[0].set(pred_W[0, :_G])
           .at[1].set(pred_W[0, _G:2 * _G]))
    nodes = _node_call(embT, gcn_W.T, w2T, gcn_b.reshape(_D, 1))
    p, q, cbs, cbd = nodes[0], nodes[1], nodes[2], nodes[3]

    sacc, qacc, dis = _scat_call(src2d, dst2d, p, q)

    bias = (b_ih + b_hh).reshape(4 * _H, 1)
    row_scale = jnp.concatenate([
        jnp.full((2 * _H, 1), 0.5, F32),
        jnp.ones((_H, 1), F32),
        jnp.full((_H, 1), 0.5, F32),
    ])
    waug = jnp.concatenate([W_hh * 0.5, W_ih, bias], axis=1) * row_scale
    waug = jnp.pad(waug, ((0, 0), (0, KA - (_H + 2)))).astype(BF16)
    wl8 = (jnp.zeros((8, _H), F32)
           .at[0].set(pred_W[0, 2 * _G:] * 0.5)).astype(BF16)
    xT = jnp.pad(x.T, ((0, 0), (0, EP - _E))).astype(BF16)
    lstm = _lstm_call(xT, waug, wl8, pred_b.reshape(1, 1))
    l2d = lstm.reshape(ER, EW)

    out2d = _final_call(src2d, dst2d, sacc, qacc, dis, p, q, cbs, cbd, l2d)
    return out2d.reshape(EP)[:_E]
